# baseline (speedup 1.0000x reference)
import functools

import jax
import jax.numpy as jnp
from jax import lax
from jax.experimental import pallas as pl
from jax.experimental.pallas import tpu as pltpu
from jax.experimental.pallas import tpu_sc as plsc

B = 1024
A = 50
D = 128
L = 16
NC, NS = 2, 16
NW = NC * NS
BPW = B // NW
APC = 10
NCHUNK = A // APC
ROWS = APC * BPW
NBUF = 2
KD = D // L
INV_A = 1.0 / A

_mesh = plsc.VectorSubcoreMesh(core_axis_name="c", subcore_axis_name="s")


@functools.partial(
    pl.kernel,
    mesh=_mesh,
    out_type=jax.ShapeDtypeStruct((B, D), jnp.float32),
    scratch_types=[
        pltpu.VMEM((A, 4 * BPW), jnp.int32),
        pltpu.VMEM((NBUF, ROWS, D), jnp.float32),
        pltpu.VMEM((BPW, D), jnp.float32),
        pltpu.SemaphoreType.DMA,
        pltpu.SemaphoreType.DMA,
    ],
)
def _cg_pool(feat_hbm, idx_hbm, out_hbm, idx_v, rows_v, out_v, sem0, sem1):
    wid = lax.axis_index("s") * NC + lax.axis_index("c")
    sems = (sem0, sem1)

    pltpu.sync_copy(idx_hbm.at[:, pl.ds((wid // 4) * 4 * BPW, 4 * BPW)], idx_v)
    col0 = (wid % 4) * BPW

    def start_gather(k, b):
        for a in range(APC):
            pltpu.async_copy(
                feat_hbm.at[idx_v.at[k * APC + a, pl.ds(col0, BPW)]],
                rows_v.at[b, pl.ds(a * BPW, BPW)],
                sems[b],
            )

    def wait_gather(k, b):
        for a in range(APC):
            pltpu.make_async_copy(
                feat_hbm.at[idx_v.at[k * APC + a, pl.ds(col0, BPW)]],
                rows_v.at[b, pl.ds(a * BPW, BPW)],
                sems[b],
            ).wait()

    for b in range(NBUF):
        start_gather(b, b)

    for k in range(NCHUNK):
        b = k % NBUF
        wait_gather(k, b)
        if k + NBUF < NCHUNK:
            start_gather(k + NBUF, b)

        def crystal_body(c, carry, _b=b, _first=(k == 0)):
            accs = [rows_v[_b, c, pl.ds(kk * L, L)] for kk in range(KD)]
            for a in range(1, APC):
                for kk in range(KD):
                    accs[kk] = accs[kk] + rows_v[_b, a * BPW + c,
                                                 pl.ds(kk * L, L)]
            for kk in range(KD):
                if _first:
                    out_v[c, pl.ds(kk * L, L)] = accs[kk]
                else:
                    plsc.addupdate(out_v.at[c, pl.ds(kk * L, L)], accs[kk])
            return carry

        lax.fori_loop(0, BPW, crystal_body, 0)

    def scale_body(c, carry):
        for kk in range(KD):
            out_v[c, pl.ds(kk * L, L)] = out_v[c, pl.ds(kk * L, L)] * INV_A
        return carry

    lax.fori_loop(0, BPW, scale_body, 0)

    pltpu.sync_copy(out_v, out_hbm.at[pl.ds(wid * BPW, BPW)])


def kernel(atom_features, atom_indices):
    idx = atom_indices
    if idx.dtype != jnp.int32:
        idx = idx.astype(jnp.int32)
    return _cg_pool(atom_features, idx.T)


# speedup vs baseline: 1.1168x; 1.1168x over previous
import functools

import jax
import jax.numpy as jnp
from jax import lax
from jax.experimental import pallas as pl
from jax.experimental.pallas import tpu as pltpu
from jax.experimental.pallas import tpu_sc as plsc

B = 1024
A = 50
D = 128
L = 16
NC, NS = 2, 16
NW = NC * NS
BPW = B // NW
NBUF = 8
KD = D // L
INV_A = 1.0 / A

_mesh = plsc.VectorSubcoreMesh(core_axis_name="c", subcore_axis_name="s")


@functools.partial(
    pl.kernel,
    mesh=_mesh,
    out_type=jax.ShapeDtypeStruct((B, D), jnp.float32),
    scratch_types=[
        pltpu.VMEM((BPW, A), jnp.int32),
        pltpu.VMEM((NBUF, A, D), jnp.float32),
        pltpu.VMEM((BPW, D), jnp.float32),
        pltpu.SemaphoreType.DMA,
        pltpu.SemaphoreType.DMA,
        pltpu.SemaphoreType.DMA,
        pltpu.SemaphoreType.DMA,
        pltpu.SemaphoreType.DMA,
        pltpu.SemaphoreType.DMA,
        pltpu.SemaphoreType.DMA,
        pltpu.SemaphoreType.DMA,
    ],
)
def _cg_pool(feat_hbm, idx_hbm, out_hbm, idx_v, rows_v, out_v,
             sem0, sem1, sem2, sem3, sem4, sem5, sem6, sem7):
    wid = lax.axis_index("s") * NC + lax.axis_index("c")
    sems = (sem0, sem1, sem2, sem3, sem4, sem5, sem6, sem7)

    pltpu.sync_copy(idx_hbm.at[pl.ds(wid * BPW, BPW)], idx_v)

    def start_gather(c, b):
        pltpu.async_copy(feat_hbm.at[idx_v.at[c]], rows_v.at[b], sems[b])

    def wait_gather(c, b):
        pltpu.make_async_copy(
            feat_hbm.at[idx_v.at[c]], rows_v.at[b], sems[b]
        ).wait()

    for b in range(NBUF):
        start_gather(b, b)

    def compute_crystal(c, b):
        def row_body(r, accs):
            return tuple(
                accs[k] + rows_v[b, r, pl.ds(k * L, L)]
                for k in range(KD)
            )

        accs = lax.fori_loop(
            0, A, row_body,
            tuple(jnp.zeros((L,), jnp.float32) for _ in range(KD)),
            unroll=5,
        )
        for k in range(KD):
            out_v[c, pl.ds(k * L, L)] = accs[k] * INV_A

    def chunk_body(o, carry):
        for b in range(NBUF):
            c = o * NBUF + b
            wait_gather(c, b)

            @pl.when(c + NBUF < BPW)
            def _start_next():
                start_gather(c + NBUF, b)

            compute_crystal(c, b)
        return carry

    lax.fori_loop(0, BPW // NBUF, chunk_body, 0)

    pltpu.sync_copy(out_v, out_hbm.at[pl.ds(wid * BPW, BPW)])


def kernel(atom_features, atom_indices):
    idx = atom_indices
    if idx.dtype != jnp.int32:
        idx = idx.astype(jnp.int32)
    return _cg_pool(atom_features, idx)


# speedup vs baseline: 1.1600x; 1.0387x over previous
import functools

import jax
import jax.numpy as jnp
from jax import lax
from jax.experimental import pallas as pl
from jax.experimental.pallas import tpu as pltpu
from jax.experimental.pallas import tpu_sc as plsc

B = 1024
A = 50
D = 128
L = 16
NC, NS = 2, 16
NW = NC * NS
BPW = B // NW
NBUF = 8
KD = D // L
INV_A = 1.0 / A

_mesh = plsc.VectorSubcoreMesh(core_axis_name="c", subcore_axis_name="s")


@functools.partial(
    pl.kernel,
    mesh=_mesh,
    out_type=jax.ShapeDtypeStruct((B, D), jnp.float32),
    scratch_types=[
        pltpu.VMEM((BPW, A), jnp.int32),
        pltpu.VMEM((NBUF, A, D), jnp.float32),
        pltpu.VMEM((BPW, D), jnp.float32),
        pltpu.SemaphoreType.DMA,
        pltpu.SemaphoreType.DMA,
        pltpu.SemaphoreType.DMA,
        pltpu.SemaphoreType.DMA,
        pltpu.SemaphoreType.DMA,
        pltpu.SemaphoreType.DMA,
        pltpu.SemaphoreType.DMA,
        pltpu.SemaphoreType.DMA,
    ],
)
def _cg_pool(feat_hbm, idx_hbm, out_hbm, idx_v, rows_v, out_v,
             sem0, sem1, sem2, sem3, sem4, sem5, sem6, sem7):
    wid = lax.axis_index("s") * NC + lax.axis_index("c")
    sems = (sem0, sem1, sem2, sem3, sem4, sem5, sem6, sem7)

    pltpu.sync_copy(idx_hbm.at[pl.ds(wid * BPW, BPW)], idx_v)

    def start_gather(c, b):
        pltpu.async_copy(feat_hbm.at[idx_v.at[c]], rows_v.at[b], sems[b])

    def wait_gather(c, b):
        pltpu.make_async_copy(
            feat_hbm.at[idx_v.at[c]], rows_v.at[b], sems[b]
        ).wait()

    for b in range(NBUF):
        start_gather(b, b)

    def compute_crystal(c, b):
        def row_body(r, accs):
            return tuple(
                accs[k] + rows_v[b, r, pl.ds(k * L, L)]
                for k in range(KD)
            )

        accs = lax.fori_loop(
            0, A, row_body,
            tuple(jnp.zeros((L,), jnp.float32) for _ in range(KD)),
        )
        for k in range(KD):
            out_v[c, pl.ds(k * L, L)] = accs[k] * INV_A

    def chunk_body(o, carry):
        for b in range(NBUF):
            c = o * NBUF + b
            wait_gather(c, b)

            @pl.when(c + NBUF < BPW)
            def _start_next():
                start_gather(c + NBUF, b)

            compute_crystal(c, b)
        return carry

    lax.fori_loop(0, BPW // NBUF, chunk_body, 0)

    pltpu.sync_copy(out_v, out_hbm.at[pl.ds(wid * BPW, BPW)])


def kernel(atom_features, atom_indices):
    idx = atom_indices
    if idx.dtype != jnp.int32:
        idx = idx.astype(jnp.int32)
    return _cg_pool(atom_features, idx)
